# baseline (speedup 1.0000x reference)
import functools

import jax
import jax.numpy as jnp
from jax import lax
from jax.experimental import pallas as pl
from jax.experimental.pallas import tpu as pltpu
from jax.experimental.pallas import tpu_sc as plsc

N = 100000
D = 128
V = 10
NC, NS = 2, 16
NW = NC * NS
CHUNK = 112
NCHUNKS = (N + CHUNK - 1) // CHUNK
T = (NCHUNKS + NW - 1) // NW
SPAN = T * CHUNK
NBUF = 6
SKEW = NBUF // 2


@functools.lru_cache(maxsize=1)
def _build():
    @functools.partial(
        pl.kernel,
        out_type=jax.ShapeDtypeStruct((N, D), jnp.float32),
        mesh=plsc.VectorSubcoreMesh(core_axis_name="c", subcore_axis_name="s"),
        scratch_types=[
            pltpu.VMEM_SHARED((V, D), jnp.float32),
            pltpu.VMEM((SPAN,), jnp.int32),
        ]
        + [pltpu.VMEM((CHUNK, D), jnp.float32) for _ in range(NBUF)]
        + [pltpu.SemaphoreType.DMA for _ in range(2 * NBUF + 2)],
    )
    def _lookup(idx_hbm, tab_hbm, out_hbm, tab_sh, idx_all, *bufs_and_sems):
        rows = bufs_and_sems[:NBUF]
        gsem = bufs_and_sems[NBUF:2 * NBUF]
        wsem = bufs_and_sems[2 * NBUF:3 * NBUF]
        isem0, isem1 = bufs_and_sems[3 * NBUF:]

        cid = lax.axis_index("c")
        sid = lax.axis_index("s")
        wid = sid * NC + cid

        span = jnp.minimum((wid * NCHUNKS) // NW * CHUNK, N - SPAN)

        idx0 = pltpu.async_copy(
            idx_hbm.at[pl.ds(span, CHUNK)], idx_all.at[pl.ds(0, CHUNK)], isem0
        )
        idx_rest = pltpu.async_copy(
            idx_hbm.at[pl.ds(span + CHUNK, SPAN - CHUNK)],
            idx_all.at[pl.ds(CHUNK, SPAN - CHUNK)],
            isem1,
        )

        @pl.when(sid == 0)
        def _():
            pltpu.sync_copy(tab_hbm, tab_sh)

        plsc.subcore_barrier()

        def start_gather(i):
            b = i % NBUF
            return pltpu.async_copy(
                tab_sh.at[idx_all.at[pl.ds(i * CHUNK, CHUNK)]], rows[b], gsem[b]
            )

        def start_write(i):
            b = i % NBUF
            return pltpu.async_copy(
                rows[b], out_hbm.at[pl.ds(span + i * CHUNK, CHUNK)], wsem[b]
            )

        gd = [None] * T
        wd = [None] * T
        idx0.wait()
        gd[0] = start_gather(0)
        idx_rest.wait()
        for j in range(1, min(NBUF, T)):
            gd[j] = start_gather(j)
        for i in range(T):
            if i >= SKEW and i + SKEW < T:
                wd[i - SKEW].wait()
                gd[i + SKEW] = start_gather(i + SKEW)
            gd[i].wait()
            wd[i] = start_write(i)
        for i in range(max(0, T - NBUF), T):
            if wd[i] is not None:
                wd[i].wait()

    return _lookup


def kernel(atomic_num, embedding):
    idx = atomic_num.astype(jnp.int32)
    return _build()(idx, embedding)


# speedup vs baseline: 1.0287x; 1.0287x over previous
import functools

import jax
import jax.numpy as jnp
from jax import lax
from jax.experimental import pallas as pl
from jax.experimental.pallas import tpu as pltpu
from jax.experimental.pallas import tpu_sc as plsc

N = 100000
D = 128
V = 10
NC, NS = 2, 16
NW = NC * NS
CHUNK = 112
NCHUNKS = (N + CHUNK - 1) // CHUNK
T = (NCHUNKS + NW - 1) // NW
SPAN = T * CHUNK
NBUF = 4


@functools.lru_cache(maxsize=1)
def _build():
    @functools.partial(
        pl.kernel,
        out_type=jax.ShapeDtypeStruct((N, D), jnp.float32),
        mesh=plsc.VectorSubcoreMesh(core_axis_name="c", subcore_axis_name="s"),
        scratch_types=[
            pltpu.VMEM_SHARED((V, D), jnp.float32),
            pltpu.VMEM((SPAN,), jnp.int32),
        ]
        + [pltpu.VMEM((CHUNK, D), jnp.float32) for _ in range(NBUF)]
        + [pltpu.SemaphoreType.DMA for _ in range(2 * NBUF + 2)],
    )
    def _lookup(idx_hbm, tab_hbm, out_hbm, tab_sh, idx_all, *bufs_and_sems):
        rows = bufs_and_sems[:NBUF]
        gsem = bufs_and_sems[NBUF:2 * NBUF]
        wsem = bufs_and_sems[2 * NBUF:3 * NBUF]
        isem0, isem1 = bufs_and_sems[3 * NBUF:]

        cid = lax.axis_index("c")
        sid = lax.axis_index("s")
        wid = sid * NC + cid

        span = jnp.minimum((wid * NCHUNKS) // NW * CHUNK, N - SPAN)

        idx0 = pltpu.async_copy(
            idx_hbm.at[pl.ds(span, CHUNK)], idx_all.at[pl.ds(0, CHUNK)], isem0
        )
        idx_rest = pltpu.async_copy(
            idx_hbm.at[pl.ds(span + CHUNK, SPAN - CHUNK)],
            idx_all.at[pl.ds(CHUNK, SPAN - CHUNK)],
            isem1,
        )

        @pl.when(sid == 0)
        def _():
            pltpu.sync_copy(tab_hbm, tab_sh)

        plsc.subcore_barrier()

        def start_gather(i, k):
            return pltpu.async_copy(
                tab_sh.at[idx_all.at[pl.ds(i * CHUNK, CHUNK)]], rows[k], gsem[k]
            )

        def wait_gather(k):
            pltpu.make_async_copy(out_hbm.at[pl.ds(0, CHUNK)], rows[k],
                                  gsem[k]).wait()

        def wait_write(k):
            pltpu.make_async_copy(rows[k], out_hbm.at[pl.ds(0, CHUNK)],
                                  wsem[k]).wait()

        idx0.wait()
        start_gather(0, 0)
        idx_rest.wait()
        for k in range(1, NBUF):
            start_gather(k, k)
        ngroups = T // NBUF

        def group(g, carry):
            for k in range(NBUF):
                i = g * NBUF + k
                wait_gather(k)
                pltpu.async_copy(
                    rows[k], out_hbm.at[pl.ds(span + i * CHUNK, CHUNK)],
                    wsem[k],
                )
            for k in range(NBUF):
                wait_write(k)

                @pl.when(g < ngroups - 1)
                def _(k=k):
                    start_gather((g + 1) * NBUF + k, k)

            return carry

        lax.fori_loop(0, ngroups, group, 0, unroll=False)

    return _lookup


def kernel(atomic_num, embedding):
    idx = atomic_num.astype(jnp.int32)
    return _build()(idx, embedding)
